# baseline (speedup 1.0000x reference)
import functools

import jax
import jax.numpy as jnp
from jax import lax
from jax.experimental import pallas as pl
from jax.experimental.pallas import tpu as pltpu
from jax.experimental.pallas import tpu_sc as plsc

N_GRAPHS = 1024
BLK = 4096
NC = 2
NS = 16
NW = NC * NS
ACC_ROWS = 1152
SUB = 128
LCH = 128
NBUF = 4


def _dense_kernel(h_ref, wcat_ref, b1_ref, w2_ref, b2_ref, bp_ref, out_ref):
    h = h_ref[...].astype(jnp.bfloat16)
    x = jnp.dot(h, wcat_ref[...],
                preferred_element_type=jnp.float32)
    feat = x[:, :128] + bp_ref[...]
    hid = jnp.maximum(x[:, 128:192] + b1_ref[...], 0.0)
    logit = jnp.dot(hid.astype(jnp.bfloat16), w2_ref[...],
                    preferred_element_type=jnp.float32) + b2_ref[0, 0]
    gate = jax.nn.sigmoid(logit)
    out_ref[...] = gate * feat


def _make_sc_segsum(n_pad):
    rows_per_w = n_pad // NW
    n_loads = rows_per_w // LCH
    acc_slab = ACC_ROWS // NS
    out_slab = N_GRAPHS // NS
    mesh = plsc.VectorSubcoreMesh(core_axis_name="c", subcore_axis_name="s",
                                  num_cores=NC)

    @functools.partial(
        pl.kernel, mesh=mesh,
        out_type=jax.ShapeDtypeStruct((NC, N_GRAPHS, 128), jnp.float32),
        scratch_types=[
            pltpu.VMEM((rows_per_w // SUB, SUB), jnp.int32),
            pltpu.VMEM((NBUF * LCH, 128), jnp.float32),
            pltpu.VMEM_SHARED((ACC_ROWS, 128), jnp.float32),
            pltpu.SemaphoreType.DMA,
            pltpu.SemaphoreType.DMA,
            pltpu.SemaphoreType.DMA,
            pltpu.SemaphoreType.DMA,
            pltpu.SemaphoreType.DMA,
            pltpu.SemaphoreType.DMA,
            pltpu.SemaphoreType.DMA,
            pltpu.SemaphoreType.DMA,
        ],
    )
    def segsum(gated_hbm, seg2d_hbm, zeros_hbm, out_hbm, idx_v, rows_v,
               acc_smem, sem0, sem1, sem2, sem3, sem4, sem5, sem6, sem7):
        sems = (sem0, sem1, sem2, sem3)
        ssems = (sem4, sem5, sem6, sem7)
        c = lax.axis_index("c")
        s = lax.axis_index("s")
        wid = c * NS + s
        w_base = wid * rows_per_w

        pltpu.sync_copy(seg2d_hbm.at[wid], idx_v)
        pltpu.sync_copy(zeros_hbm.at[pl.ds(s * acc_slab, acc_slab), :],
                        acc_smem.at[pl.ds(s * acc_slab, acc_slab), :])
        plsc.subcore_barrier()

        def start_load(k):
            b = k % NBUF
            return pltpu.async_copy(
                gated_hbm.at[pl.ds(w_base + k * LCH, LCH), :],
                rows_v.at[pl.ds(b * LCH, LCH), :], sems[b])

        inflight = {}
        scatters = {}
        for k in range(min(NBUF - 1, n_loads)):
            inflight[k] = start_load(k)
        for k in range(n_loads):
            inflight.pop(k).wait()
            b = k % NBUF
            scatters[k] = pltpu.async_copy(
                rows_v.at[pl.ds(b * LCH, LCH), :],
                acc_smem.at[idx_v.at[k]], ssems[b], add=True)
            m = k + NBUF - 1
            if m < n_loads:
                if m - NBUF in scatters:
                    scatters.pop(m - NBUF).wait()
                inflight[m] = start_load(m)
        for k in sorted(scatters):
            scatters.pop(k).wait()
        plsc.subcore_barrier()

        pltpu.sync_copy(acc_smem.at[pl.ds(s * out_slab, out_slab), :],
                        out_hbm.at[c, pl.ds(s * out_slab, out_slab), :])

    return segsum


@jax.jit
def kernel(h_nodes, batch, W1, b1, W2, b2, Wp, bp):
    n, d = h_nodes.shape
    out_dim = Wp.shape[0]
    hidden = W1.shape[0]
    n_pad = -(-n // (NW * LCH)) * (NW * LCH)
    nblk = n_pad // BLK
    assert (nblk - 1) * BLK < n
    seg2d = jnp.pad(batch.astype(jnp.int32), (0, n_pad - n),
                    constant_values=N_GRAPHS).reshape(
                        NW, n_pad // (NW * SUB), SUB)

    wcat = jnp.zeros((d, 256), jnp.float32)
    wcat = wcat.at[:, :out_dim].set(Wp.T).at[:, 128:128 + hidden].set(W1.T)
    wcat = wcat.astype(jnp.bfloat16)
    b1r = b1.reshape(1, hidden)
    w2t = jnp.tile(W2.T, (1, 128)).astype(jnp.bfloat16)
    b2r = b2.reshape(1, 1)
    bpr = bp.reshape(1, out_dim)

    gated = pl.pallas_call(
        _dense_kernel,
        grid=(nblk,),
        in_specs=[
            pl.BlockSpec((BLK, d), lambda i: (i, 0)),
            pl.BlockSpec((d, 256), lambda i: (0, 0)),
            pl.BlockSpec((1, hidden), lambda i: (0, 0)),
            pl.BlockSpec((hidden, 128), lambda i: (0, 0)),
            pl.BlockSpec((1, 1), lambda i: (0, 0)),
            pl.BlockSpec((1, out_dim), lambda i: (0, 0)),
        ],
        out_specs=pl.BlockSpec((BLK, out_dim), lambda i: (i, 0)),
        out_shape=jax.ShapeDtypeStruct((n_pad, out_dim), jnp.float32),
    )(h_nodes, wcat, b1r, w2t, b2r, bpr)

    zeros = jnp.zeros((ACC_ROWS, 128), jnp.float32)
    partials = _make_sc_segsum(n_pad)(gated, seg2d, zeros)
    return partials[0] + partials[1]


# speedup vs baseline: 1.0584x; 1.0584x over previous
import functools

import jax
import jax.numpy as jnp
from jax import lax
from jax.experimental import pallas as pl
from jax.experimental.pallas import tpu as pltpu
from jax.experimental.pallas import tpu_sc as plsc

N_GRAPHS = 1024
BLK = 4096
NC = 2
NS = 16
NW = NC * NS
ACC_ROWS = 1152
SUB = 128
LCH = 128
NBUF = 4


def _dense_kernel(h_ref, wcat_ref, b1_ref, w2_ref, b2_ref, bp_ref, out_ref):
    h = h_ref[...].astype(jnp.bfloat16)
    x = jnp.dot(h, wcat_ref[...],
                preferred_element_type=jnp.float32)
    feat = x[:, :128] + bp_ref[...]
    hid = jnp.maximum(x[:, 128:192] + b1_ref[...], 0.0)
    logit = jnp.dot(hid.astype(jnp.bfloat16), w2_ref[...],
                    preferred_element_type=jnp.float32) + b2_ref[0, 0]
    gate = jax.nn.sigmoid(logit)
    out_ref[...] = gate * feat


def _make_sc_segsum(n_pad):
    rows_per_w = n_pad // NW
    n_loads = rows_per_w // LCH
    acc_slab = ACC_ROWS // NS
    out_slab = N_GRAPHS // NS
    mesh = plsc.VectorSubcoreMesh(core_axis_name="c", subcore_axis_name="s",
                                  num_cores=NC)

    @functools.partial(
        pl.kernel, mesh=mesh,
        out_type=jax.ShapeDtypeStruct((NC, N_GRAPHS, 128), jnp.float32),
        scratch_types=[
            pltpu.VMEM((rows_per_w // SUB, SUB), jnp.int32),
            pltpu.VMEM((NBUF * LCH, 128), jnp.float32),
            pltpu.VMEM_SHARED((ACC_ROWS, 128), jnp.float32),
            pltpu.SemaphoreType.DMA,
            pltpu.SemaphoreType.DMA,
            pltpu.SemaphoreType.DMA,
            pltpu.SemaphoreType.DMA,
            pltpu.SemaphoreType.DMA,
            pltpu.SemaphoreType.DMA,
            pltpu.SemaphoreType.DMA,
            pltpu.SemaphoreType.DMA,
        ],
    )
    def segsum(gated_hbm, seg2d_hbm, zeros_hbm, out_hbm, idx_v, rows_v,
               acc_smem, sem0, sem1, sem2, sem3, sem4, sem5, sem6, sem7):
        sems = (sem0, sem1, sem2, sem3)
        ssems = (sem4, sem5, sem6, sem7)
        c = lax.axis_index("c")
        s = lax.axis_index("s")
        wid = c * NS + s
        w_base = wid * rows_per_w

        pltpu.sync_copy(seg2d_hbm.at[wid], idx_v)
        pltpu.sync_copy(zeros_hbm.at[pl.ds(s * acc_slab, acc_slab), :],
                        acc_smem.at[pl.ds(s * acc_slab, acc_slab), :])
        plsc.subcore_barrier()

        def start_load(k):
            b = k % NBUF
            return pltpu.async_copy(
                gated_hbm.at[pl.ds(w_base + k * LCH, LCH), :],
                rows_v.at[pl.ds(b * LCH, LCH), :], sems[b])

        inflight = {}
        scatters = {}
        for k in range(min(NBUF - 1, n_loads)):
            inflight[k] = start_load(k)
        for k in range(n_loads):
            inflight.pop(k).wait()
            b = k % NBUF
            scatters[k] = pltpu.async_copy(
                rows_v.at[pl.ds(b * LCH, LCH), :],
                acc_smem.at[idx_v.at[k]], ssems[b], add=True)
            m = k + NBUF - 1
            if m < n_loads:
                if m - NBUF in scatters:
                    scatters.pop(m - NBUF).wait()
                inflight[m] = start_load(m)
        for k in sorted(scatters):
            scatters.pop(k).wait()
        plsc.subcore_barrier()

        pltpu.sync_copy(acc_smem.at[pl.ds(s * out_slab, out_slab), :],
                        out_hbm.at[c, pl.ds(s * out_slab, out_slab), :])

    return segsum


@jax.jit
def kernel(h_nodes, batch, W1, b1, W2, b2, Wp, bp):
    n, d = h_nodes.shape
    out_dim = Wp.shape[0]
    hidden = W1.shape[0]
    n_pad = -(-n // (NW * LCH)) * (NW * LCH)
    nblk = n_pad // BLK
    assert (nblk - 1) * BLK < n
    seg_flat = jnp.pad(batch.astype(jnp.int32), (0, n_pad - n),
                       constant_values=N_GRAPHS)

    wcat = jnp.zeros((d, 256), jnp.float32)
    wcat = wcat.at[:, :out_dim].set(Wp.T).at[:, 128:128 + hidden].set(W1.T)
    wcat = wcat.astype(jnp.bfloat16)
    b1r = b1.reshape(1, hidden)
    w2t = jnp.tile(W2.T, (1, 128)).astype(jnp.bfloat16)
    b2r = b2.reshape(1, 1)
    bpr = bp.reshape(1, out_dim)

    blk_a = -(-nblk // 2)
    halves = [(0, blk_a), (blk_a, nblk - blk_a)]
    zeros = jnp.zeros((ACC_ROWS, 128), jnp.float32)

    partial_sums = []
    for off_blocks, nb in halves:
        n_half = nb * BLK
        off_rows = off_blocks * BLK
        gated = pl.pallas_call(
            _dense_kernel,
            grid=(nb,),
            in_specs=[
                pl.BlockSpec((BLK, d),
                             lambda i, o=off_blocks: (i + o, 0)),
                pl.BlockSpec((d, 256), lambda i: (0, 0)),
                pl.BlockSpec((1, hidden), lambda i: (0, 0)),
                pl.BlockSpec((hidden, 128), lambda i: (0, 0)),
                pl.BlockSpec((1, 1), lambda i: (0, 0)),
                pl.BlockSpec((1, out_dim), lambda i: (0, 0)),
            ],
            out_specs=pl.BlockSpec((BLK, out_dim), lambda i: (i, 0)),
            out_shape=jax.ShapeDtypeStruct((n_half, out_dim), jnp.float32),
        )(h_nodes, wcat, b1r, w2t, b2r, bpr)

        seg2d = seg_flat[off_rows:off_rows + n_half].reshape(
            NW, n_half // (NW * SUB), SUB)
        partial_sums.append(_make_sc_segsum(n_half)(gated, seg2d, zeros))

    pa, pb = partial_sums
    return pa[0] + pa[1] + pb[0] + pb[1]
